# baseline (speedup 1.0000x reference)
import functools

import jax
import jax.numpy as jnp
from jax import lax
from jax.experimental import pallas as pl
from jax.experimental.pallas import tpu as pltpu
from jax.experimental.pallas import tpu_sc as plsc

N = 50000
E = 800000
B = 64
NC, NS = 2, 16
NW = NC * NS
EPW = 25600
EPAD = NW * EPW
CA = 128
NPAD = 50048
STRIPE = NPAD // NS
NEG = -1e30

_SC_PARAMS = pltpu.CompilerParams(use_tc_tiling_on_sc=False)


def _tree_sum(v):
    idx = lax.iota(jnp.int32, 16)
    for sh in (8, 4, 2, 1):
        v = v + v[(idx + sh) % 16]
    return v


def _mm_kernel(x_ref, w_ref, b_ref, o_ref, *, act):
    y = jnp.dot(x_ref[...], w_ref[...], preferred_element_type=jnp.float32)
    y = y + b_ref[...][None, :]
    if act == "relu":
        y = jnp.maximum(y, 0.0)
    o_ref[...] = y


def _mm(x, W, b, act=None, block_m=512):
    M, K = x.shape
    Nf = W.shape[1]
    Mp = ((M + block_m - 1) // block_m) * block_m
    if Mp != M:
        x = jnp.pad(x, ((0, Mp - M), (0, 0)))
    out = pl.pallas_call(
        functools.partial(_mm_kernel, act=act),
        grid=(Mp // block_m,),
        in_specs=[
            pl.BlockSpec((block_m, K), lambda i: (i, 0)),
            pl.BlockSpec((K, Nf), lambda i: (0, 0)),
            pl.BlockSpec((Nf,), lambda i: (0,)),
        ],
        out_specs=pl.BlockSpec((block_m, Nf), lambda i: (i, 0)),
        out_shape=jax.ShapeDtypeStruct((Mp, Nf), jnp.float32),
    )(x, W, b)
    return out[:M]


def _pass_a(xl, xr, att, sd, D):
    mesh = plsc.VectorSubcoreMesh(core_axis_name="c", subcore_axis_name="s")

    NCH = EPW // CA

    @functools.partial(
        pl.kernel,
        out_type=(
            jax.ShapeDtypeStruct((EPAD,), jnp.float32),
            jax.ShapeDtypeStruct((NW * 16,), jnp.float32),
        ),
        mesh=mesh,
        scratch_types=(
            pltpu.VMEM((2, CA), jnp.int32),
            pltpu.VMEM((2, CA), jnp.int32),
            pltpu.VMEM((CA, D), jnp.float32),
            pltpu.VMEM((CA, D), jnp.float32),
            pltpu.VMEM((CA, D), jnp.float32),
            pltpu.VMEM((CA, D), jnp.float32),
            pltpu.VMEM((CA,), jnp.float32),
            pltpu.VMEM((D,), jnp.float32),
            pltpu.SemaphoreType.DMA,
            pltpu.SemaphoreType.DMA,
            pltpu.SemaphoreType.DMA,
            pltpu.SemaphoreType.DMA,
            pltpu.SemaphoreType.DMA,
            pltpu.SemaphoreType.DMA,
        ),
        compiler_params=_SC_PARAMS,
    )
    def k(xl_h, xr_h, att_h, sd_h, lo_h, mx_h,
          sd0, sd1, gl0, gr0, gl1, gr1,
          lov, attv, semg0, semg1, semg2, semg3, semt0, semt1):
        c = lax.axis_index("c")
        s = lax.axis_index("s")
        wid = s * NC + c
        base = wid * EPW
        pltpu.sync_copy(att_h, attv)
        iota = lax.iota(jnp.int32, 16)
        attk = [attv[pl.ds(k * 16, 16)] for k in range(D // 16)]
        bufs = ((sd0, gl0, gr0, semg0, semg1, semt0),
                (sd1, gl1, gr1, semg2, semg3, semt1))

        def stage(i, p):
            sdb = bufs[p][0]
            pltpu.async_copy(sd_h.at[wid * NCH + i], sdb, bufs[p][5])

        def stage_wait(p):
            sdb = bufs[p][0]
            pltpu.make_async_copy(sd_h.at[wid * NCH], sdb, bufs[p][5]).wait()

        def gissue(p):
            sdb, gl, gr, sl_, sr_, _ = bufs[p]
            pltpu.async_copy(xl_h.at[sdb.at[0]], gl, sl_)
            pltpu.async_copy(xr_h.at[sdb.at[1]], gr, sr_)

        def drain(p):
            sdb, gl, gr, sl_, sr_, _ = bufs[p]
            pltpu.make_async_copy(xl_h.at[sdb.at[0]], gl, sl_).wait()
            pltpu.make_async_copy(xr_h.at[sdb.at[1]], gr, sr_).wait()

        stage(0, 0)
        stage(1, 1)
        stage_wait(0)
        gissue(0)

        def outer(cj, mx):
            for p in range(2):
                i = 2 * cj + p
                eb = base + i * CA
                nxt = i + 1

                @pl.when(nxt < NCH)
                def _():
                    stage_wait(1 - p)
                    gissue(1 - p)

                drain(p)

                @pl.when(i + 2 < NCH)
                def _():
                    stage(i + 2, p)

                _, gl, gr, _, _, _ = bufs[p]

                def grp_body(g, mx):
                    out = jnp.full((16,), NEG, jnp.float32)
                    for e in range(16):
                        row = g * 16 + e
                        acc = jnp.zeros((16,), jnp.float32)
                        for kk in range(D // 16):
                            t = gl[row, pl.ds(kk * 16, 16)] + gr[row, pl.ds(kk * 16, 16)]
                            t = jnp.maximum(t, 0.2 * t)
                            acc = acc + attk[kk] * t
                        out = jnp.where(iota == e, _tree_sum(acc), out)
                    eids = eb + g * 16 + iota
                    out = jnp.where(eids < E, out, NEG)
                    lov[pl.ds(g * 16, 16)] = out
                    return jnp.maximum(mx, out)

                mx = lax.fori_loop(0, CA // 16, grp_body, mx)
                pltpu.sync_copy(lov, lo_h.at[pl.ds(eb, CA)])
            return mx

        mx = lax.fori_loop(0, NCH // 2, outer,
                           jnp.full((16,), NEG, jnp.float32))
        lov[pl.ds(0, 16)] = mx
        pltpu.sync_copy(lov.at[pl.ds(0, 16)], mx_h.at[pl.ds(wid * 16, 16)])

    return k(xl, xr, att, sd)


def _pass_c(tables, pk, csplat, zstripe):
    P = len(tables) + 1
    CB = 2 * CA
    NCH = EPW // CB
    mesh = plsc.VectorSubcoreMesh(core_axis_name="c", subcore_axis_name="s")

    @functools.partial(
        pl.kernel,
        out_type=tuple(jax.ShapeDtypeStruct((NC, NPAD, 32), jnp.float32)
                       for _ in range(P)),
        mesh=mesh,
        scratch_types=(
            pltpu.VMEM((6, CA), jnp.int32),
            pltpu.VMEM((6, CA), jnp.int32),
            pltpu.VMEM((6, CA), jnp.int32),
            pltpu.VMEM((6, CA), jnp.int32),
            pltpu.VMEM((CB, 32), jnp.float32),
            pltpu.VMEM((CB, 32), jnp.float32),
            pltpu.VMEM((CB,), jnp.float32),
            pltpu.VMEM((16,), jnp.float32),
            pltpu.VMEM_SHARED((NPAD, 32), jnp.float32),
            pltpu.SemaphoreType.DMA,
            pltpu.SemaphoreType.DMA,
            pltpu.SemaphoreType.DMA,
            pltpu.SemaphoreType.DMA,
            pltpu.SemaphoreType.DMA,
            pltpu.SemaphoreType.DMA,
            pltpu.SemaphoreType.DMA,
            pltpu.SemaphoreType.DMA,
        ),
        compiler_params=_SC_PARAMS,
    )
    def k(*refs):
        t_hs = refs[:P - 1]
        pk_h, c_h, z_h = refs[P - 1:P + 2]
        out_hs = refs[P + 2:2 * P + 2]
        (pb0, pb1, pb2, pb3, rows0, rows1, abuf, cv, slab,
         st0, st1, st2, st3, sg0, sg1, sc0, sc1) = refs[2 * P + 2:]
        c = lax.axis_index("c")
        s = lax.axis_index("s")
        wid = s * NC + c
        pltpu.sync_copy(c_h, cv)
        cvec = cv[...]
        iota = lax.iota(jnp.int32, 16)
        pbs = (pb0, pb1, pb2, pb3)
        sts = (st0, st1, st2, st3)
        rws = (rows0, rows1)
        sgs = (sg0, sg1)
        scs = (sc0, sc1)

        for p in range(P):
            is_den = p == P - 1
            pltpu.sync_copy(z_h, slab.at[pl.ds(s * STRIPE, STRIPE)])
            plsc.subcore_barrier()

            def stage(i, q):
                pltpu.async_copy(pk_h.at[wid * NCH + i], pbs[q], sts[q])

            def stage_wait(q):
                pltpu.make_async_copy(pk_h.at[wid * NCH], pbs[q], sts[q]).wait()

            def gissue(i, q, pa):
                pltpu.async_copy(t_hs[p].at[pbs[q].at[0]],
                                 rws[pa].at[pl.ds(0, CA)], sgs[pa])
                pltpu.async_copy(t_hs[p].at[pbs[q].at[1]],
                                 rws[pa].at[pl.ds(CA, CA)], sgs[pa])

            def gwait(q, pa):
                pltpu.make_async_copy(
                    t_hs[p].at[pbs[q].at[0]],
                    rws[pa].at[pl.ds(0, CA)], sgs[pa]).wait()
                pltpu.make_async_copy(
                    t_hs[p].at[pbs[q].at[1]],
                    rws[pa].at[pl.ds(CA, CA)], sgs[pa]).wait()

            def scissue(q, pa):
                pltpu.async_copy(rws[pa].at[pl.ds(0, CA)],
                                 slab.at[pbs[q].at[2]], scs[pa], add=True)
                pltpu.async_copy(rws[pa].at[pl.ds(CA, CA)],
                                 slab.at[pbs[q].at[3]], scs[pa], add=True)

            def scwait(q, pa):
                pltpu.make_async_copy(
                    rws[pa].at[pl.ds(0, CA)],
                    slab.at[pbs[q].at[2]], scs[pa]).wait()
                pltpu.make_async_copy(
                    rws[pa].at[pl.ds(CA, CA)],
                    slab.at[pbs[q].at[3]], scs[pa]).wait()

            stage(0, 0)
            stage(1, 1)
            stage_wait(0)
            if not is_den:
                gissue(0, 0, 0)

            def outer(cj, _):
                for u in range(4):
                    i4 = 4 * cj + u
                    q = u
                    pa = u % 2
                    qn = (u + 1) % 4
                    qnn = (u + 2) % 4
                    nxt = i4 + 1

                    @pl.when(nxt < NCH)
                    def _():
                        stage_wait(qn)

                    @pl.when(i4 >= 1)
                    def _():
                        scwait((u - 1) % 4, 1 - pa)

                    if not is_den:
                        @pl.when(nxt < NCH)
                        def _():
                            gissue(nxt, qn, 1 - pa)

                        gwait(q, pa)

                    @pl.when(i4 + 2 < NCH)
                    def _():
                        stage(i4 + 2, qnn)

                    pb = pbs[q]
                    rows = rws[pa]

                    def agrp(g, _):
                        lv = lax.bitcast_convert_type(
                            pb[4 + g // 8, pl.ds((g % 8) * 16, 16)],
                            jnp.float32)
                        a = jnp.exp(lv - cvec)
                        abuf[pl.ds(g * 16, 16)] = a
                        return 0

                    lax.fori_loop(0, CB // 16, agrp, 0)

                    if is_den:
                        def dgrp(g, _):
                            av = abuf[pl.ds(g * 16, 16)]
                            for e in range(16):
                                spl = av[jnp.full((16,), e, jnp.int32)]
                                rows[g * 16 + e, pl.ds(0, 16)] = \
                                    jnp.where(iota == 0, spl, 0.0)
                                rows[g * 16 + e, pl.ds(16, 16)] = \
                                    jnp.zeros((16,), jnp.float32)
                            return 0

                        lax.fori_loop(0, CB // 16, dgrp, 0)
                    else:
                        def sgrp(g, _):
                            av = abuf[pl.ds(g * 16, 16)]
                            for e in range(16):
                                spl = av[jnp.full((16,), e, jnp.int32)]
                                r = g * 16 + e
                                rows[r, pl.ds(0, 16)] = rows[r, pl.ds(0, 16)] * spl
                                rows[r, pl.ds(16, 16)] = rows[r, pl.ds(16, 16)] * spl
                            return 0

                        lax.fori_loop(0, CB // 16, sgrp, 0)

                    scissue(q, pa)
                return 0

            lax.fori_loop(0, NCH // 4, outer, 0)
            scwait((NCH - 1) % 4, (NCH - 1) % 2)
            plsc.subcore_barrier()
            pltpu.sync_copy(slab.at[pl.ds(s * STRIPE, STRIPE)],
                            out_hs[p].at[c, pl.ds(s * STRIPE, STRIPE)])

    return k(*tables, pk, csplat, zstripe)


def _selflogit_kernel(xl_ref, xr_ref, att_ref, o_ref, m_ref):
    t = xl_ref[...] + xr_ref[...]
    t = jnp.maximum(t, 0.2 * t)
    sl = t @ att_ref[...]
    o_ref[...] = sl

    @pl.when(pl.program_id(0) == 0)
    def _():
        m_ref[...] = jnp.full((1, 128), NEG, jnp.float32)

    m_ref[...] = jnp.maximum(m_ref[...], jnp.max(sl))


def _selflogit(xl, xr, att, block_m=512):
    M, D = xl.shape
    Mp = ((M + block_m - 1) // block_m) * block_m
    if Mp != M:
        xl = jnp.pad(xl, ((0, Mp - M), (0, 0)))
        xr = jnp.pad(xr, ((0, Mp - M), (0, 0)))
    sl, m = pl.pallas_call(
        _selflogit_kernel,
        grid=(Mp // block_m,),
        in_specs=[
            pl.BlockSpec((block_m, D), lambda i: (i, 0)),
            pl.BlockSpec((block_m, D), lambda i: (i, 0)),
            pl.BlockSpec((D,), lambda i: (0,)),
        ],
        out_specs=[
            pl.BlockSpec((block_m,), lambda i: (i,)),
            pl.BlockSpec((1, 128), lambda i: (0, 0)),
        ],
        out_shape=[
            jax.ShapeDtypeStruct((Mp,), jnp.float32),
            jax.ShapeDtypeStruct((1, 128), jnp.float32),
        ],
    )(xl, xr, att)
    return sl[:M], m


def _pool_kernel(h_ref, b_ref, s_ref, c_ref):
    oh = (b_ref[...][:, None] == lax.iota(jnp.int32, B)[None, :]).astype(jnp.float32)

    @pl.when(pl.program_id(0) == 0)
    def _():
        s_ref[...] = jnp.zeros_like(s_ref)
        c_ref[...] = jnp.zeros_like(c_ref)

    s_ref[...] += jnp.dot(oh.T, h_ref[...], preferred_element_type=jnp.float32)
    c_ref[...] += jnp.sum(oh, axis=0)[None, :]


def _pool(h, batch, block_m=512):
    M = h.shape[0]
    Mp = ((M + block_m - 1) // block_m) * block_m
    if Mp != M:
        h = jnp.pad(h, ((0, Mp - M), (0, 0)))
        batch = jnp.pad(batch, (0, Mp - M), constant_values=B)
    sums, counts = pl.pallas_call(
        _pool_kernel,
        grid=(Mp // block_m,),
        in_specs=[
            pl.BlockSpec((block_m, 32), lambda i: (i, 0)),
            pl.BlockSpec((block_m,), lambda i: (i,)),
        ],
        out_specs=[
            pl.BlockSpec((B, 32), lambda i: (0, 0)),
            pl.BlockSpec((1, B), lambda i: (0, 0)),
        ],
        out_shape=[
            jax.ShapeDtypeStruct((B, 32), jnp.float32),
            jax.ShapeDtypeStruct((1, B), jnp.float32),
        ],
    )(h, batch)
    return sums / jnp.maximum(counts[0], 1.0)[:, None]


def _merge_kernel(xl_ref, sl_ref, cb_ref, b_ref, *rest, nq):
    o_refs = rest[:nq + 1]
    h_ref = rest[nq + 1]
    C = cb_ref[0, 0]
    sw = jnp.exp(sl_ref[...] - C)
    num = jnp.concatenate(
        [o[0] + o[1] for o in (r[...] for r in o_refs[:nq])], axis=1)
    den = o_refs[nq][...][0, :, 0] + o_refs[nq][...][1, :, 0]
    h = (num + sw[:, None] * xl_ref[...]) / (den + sw)[:, None] + b_ref[...][None, :]
    h_ref[...] = jnp.where(h > 0, h, jnp.exp(jnp.minimum(h, 0.0)) - 1.0)


def _merge(outs, xl, sl, C, bias, D, block_m=2048):
    nq = D // 32
    M = xl.shape[0]
    Mp = ((M + block_m - 1) // block_m) * block_m
    if Mp != M:
        xl = jnp.pad(xl, ((0, Mp - M), (0, 0)))
        sl = jnp.pad(sl, (0, Mp - M))
    cb = jnp.full((1, 128), C, jnp.float32)
    h = pl.pallas_call(
        functools.partial(_merge_kernel, nq=nq),
        grid=(Mp // block_m,),
        in_specs=[
            pl.BlockSpec((block_m, D), lambda i: (i, 0)),
            pl.BlockSpec((block_m,), lambda i: (i,)),
            pl.BlockSpec((1, 128), lambda i: (0, 0)),
            pl.BlockSpec((D,), lambda i: (0,)),
        ] + [pl.BlockSpec((2, block_m, 32), lambda i: (0, i, 0))
             for _ in range(nq + 1)],
        out_specs=pl.BlockSpec((block_m, D), lambda i: (i, 0)),
        out_shape=jax.ShapeDtypeStruct((Mp, D), jnp.float32),
    )(xl, sl, cb, bias, *outs)
    return h[:M]


def _gatv2_sc(xl, xr, att, bias, sd, srcp, dstp, zstripe, D):
    lo, wmax = _pass_a(xl, xr, att, sd, D)
    sl, smax = _selflogit(xl, xr, att)
    C = jnp.maximum(jnp.max(wmax), jnp.max(smax))
    csplat = jnp.full((16,), C, jnp.float32)
    pk = jnp.concatenate([
        srcp.reshape(-1, 2, CA),
        dstp.reshape(-1, 2, CA),
        lax.bitcast_convert_type(lo, jnp.int32).reshape(-1, 2, CA),
    ], axis=1)
    nq = D // 32
    tables = [xl[:, q * 32:(q + 1) * 32] for q in range(nq)]
    outs = _pass_c(tables, pk, csplat, zstripe)
    return _merge(outs, xl, sl, C, bias, D)


def kernel(x, edge_index, batch, observable_features, noise_factor, noisy_exp, Wl1, bl1, Wr1, br1, att1, bias1, Wl2, bl2, Wr2, br2, att2, bias2, Wo1, bo1, Wo2, bo2, Wn, bn, Wf1, bf1, Wf2, bf2, Wf3, bf3):
    srcp = jnp.pad(edge_index[0], (0, EPAD - E))
    dstp = jnp.pad(edge_index[1], (0, EPAD - E))
    sd = jnp.concatenate(
        [srcp.reshape(-1, 1, CA), dstp.reshape(-1, 1, CA)], axis=1)
    zstripe = jnp.zeros((STRIPE, 32), jnp.float32)

    xl1 = _mm(x, Wl1, bl1)
    xr1 = _mm(x, Wr1, br1)
    h = _gatv2_sc(xl1, xr1, att1, bias1, sd, srcp, dstp, zstripe, 128)
    xl2 = _mm(h, Wl2, bl2)
    xr2 = _mm(h, Wr2, br2)
    h = _gatv2_sc(xl2, xr2, att2, bias2, sd, srcp, dstp, zstripe, 32)

    circuit_embedding = _pool(h, batch)
    obs_embedding = _mm(_mm(observable_features, Wo1, bo1, act="relu"), Wo2, bo2, block_m=64)
    noise_embedding = _mm(noise_factor, Wn, bn, block_m=64)
    ne = noisy_exp.reshape(-1, 1)
    combined = jnp.concatenate([circuit_embedding, obs_embedding, noise_embedding, ne], axis=1)
    c = _mm(combined, Wf1, bf1, act="relu", block_m=64)
    c = _mm(c, Wf2, bf2, act="relu", block_m=64)
    correction = _mm(c, Wf3, bf3, block_m=64)
    return ne + correction


# speedup vs baseline: 1.0193x; 1.0193x over previous
import functools

import jax
import jax.numpy as jnp
from jax import lax
from jax.experimental import pallas as pl
from jax.experimental.pallas import tpu as pltpu
from jax.experimental.pallas import tpu_sc as plsc

N = 50000
E = 800000
B = 64
NC, NS = 2, 16
NW = NC * NS
EPW = 25600
EPAD = NW * EPW
CA = 128
NPAD = 50048
STRIPE = NPAD // NS
NEG = -1e30

_SC_PARAMS = pltpu.CompilerParams(use_tc_tiling_on_sc=False)


def _tree_sum(v):
    idx = lax.iota(jnp.int32, 16)
    for sh in (8, 4, 2, 1):
        v = v + v[(idx + sh) % 16]
    return v


def _mm_kernel(x_ref, w_ref, b_ref, o_ref, *, act):
    y = jnp.dot(x_ref[...], w_ref[...], preferred_element_type=jnp.float32)
    y = y + b_ref[...][None, :]
    if act == "relu":
        y = jnp.maximum(y, 0.0)
    o_ref[...] = y


def _mm(x, W, b, act=None, block_m=512):
    M, K = x.shape
    Nf = W.shape[1]
    Mp = ((M + block_m - 1) // block_m) * block_m
    if Mp != M:
        x = jnp.pad(x, ((0, Mp - M), (0, 0)))
    out = pl.pallas_call(
        functools.partial(_mm_kernel, act=act),
        grid=(Mp // block_m,),
        in_specs=[
            pl.BlockSpec((block_m, K), lambda i: (i, 0)),
            pl.BlockSpec((K, Nf), lambda i: (0, 0)),
            pl.BlockSpec((Nf,), lambda i: (0,)),
        ],
        out_specs=pl.BlockSpec((block_m, Nf), lambda i: (i, 0)),
        out_shape=jax.ShapeDtypeStruct((Mp, Nf), jnp.float32),
    )(x, W, b)
    return out[:M]


def _pass_a(xl, xr, att, sd, D):
    mesh = plsc.VectorSubcoreMesh(core_axis_name="c", subcore_axis_name="s")

    NCH = EPW // CA

    @functools.partial(
        pl.kernel,
        out_type=(
            jax.ShapeDtypeStruct((EPAD,), jnp.float32),
            jax.ShapeDtypeStruct((NW * 16,), jnp.float32),
        ),
        mesh=mesh,
        scratch_types=(
            pltpu.VMEM((2, CA), jnp.int32),
            pltpu.VMEM((2, CA), jnp.int32),
            pltpu.VMEM((CA, D), jnp.float32),
            pltpu.VMEM((CA, D), jnp.float32),
            pltpu.VMEM((CA, D), jnp.float32),
            pltpu.VMEM((CA, D), jnp.float32),
            pltpu.VMEM((CA,), jnp.float32),
            pltpu.VMEM((D,), jnp.float32),
            pltpu.SemaphoreType.DMA,
            pltpu.SemaphoreType.DMA,
            pltpu.SemaphoreType.DMA,
            pltpu.SemaphoreType.DMA,
            pltpu.SemaphoreType.DMA,
            pltpu.SemaphoreType.DMA,
        ),
        compiler_params=_SC_PARAMS,
    )
    def k(xl_h, xr_h, att_h, sd_h, lo_h, mx_h,
          sd0, sd1, gl0, gr0, gl1, gr1,
          lov, attv, semg0, semg1, semg2, semg3, semt0, semt1):
        c = lax.axis_index("c")
        s = lax.axis_index("s")
        wid = s * NC + c
        base = wid * EPW
        pltpu.sync_copy(att_h, attv)
        iota = lax.iota(jnp.int32, 16)
        attk = [attv[pl.ds(k * 16, 16)] for k in range(D // 16)]
        bufs = ((sd0, gl0, gr0, semg0, semg1, semt0),
                (sd1, gl1, gr1, semg2, semg3, semt1))

        def stage(i, p):
            sdb = bufs[p][0]
            pltpu.async_copy(sd_h.at[wid * NCH + i], sdb, bufs[p][5])

        def stage_wait(p):
            sdb = bufs[p][0]
            pltpu.make_async_copy(sd_h.at[wid * NCH], sdb, bufs[p][5]).wait()

        def gissue(p):
            sdb, gl, gr, sl_, sr_, _ = bufs[p]
            pltpu.async_copy(xl_h.at[sdb.at[0]], gl, sl_)
            pltpu.async_copy(xr_h.at[sdb.at[1]], gr, sr_)

        def drain(p):
            sdb, gl, gr, sl_, sr_, _ = bufs[p]
            pltpu.make_async_copy(xl_h.at[sdb.at[0]], gl, sl_).wait()
            pltpu.make_async_copy(xr_h.at[sdb.at[1]], gr, sr_).wait()

        stage(0, 0)
        stage(1, 1)
        stage_wait(0)
        gissue(0)

        def outer(cj, mx):
            for p in range(2):
                i = 2 * cj + p
                eb = base + i * CA
                nxt = i + 1

                @pl.when(nxt < NCH)
                def _():
                    stage_wait(1 - p)
                    gissue(1 - p)

                drain(p)

                @pl.when(i + 2 < NCH)
                def _():
                    stage(i + 2, p)

                _, gl, gr, _, _, _ = bufs[p]

                def grp_body(g, mx):
                    out = jnp.full((16,), NEG, jnp.float32)
                    for e in range(16):
                        row = g * 16 + e
                        acc = jnp.zeros((16,), jnp.float32)
                        for kk in range(D // 16):
                            t = gl[row, pl.ds(kk * 16, 16)] + gr[row, pl.ds(kk * 16, 16)]
                            t = jnp.maximum(t, 0.2 * t)
                            acc = acc + attk[kk] * t
                        out = jnp.where(iota == e, _tree_sum(acc), out)
                    eids = eb + g * 16 + iota
                    out = jnp.where(eids < E, out, NEG)
                    lov[pl.ds(g * 16, 16)] = out
                    return jnp.maximum(mx, out)

                mx = lax.fori_loop(0, CA // 16, grp_body, mx)
                pltpu.sync_copy(lov, lo_h.at[pl.ds(eb, CA)])
            return mx

        mx = lax.fori_loop(0, NCH // 2, outer,
                           jnp.full((16,), NEG, jnp.float32))
        lov[pl.ds(0, 16)] = mx
        pltpu.sync_copy(lov.at[pl.ds(0, 16)], mx_h.at[pl.ds(wid * 16, 16)])

    return k(xl, xr, att, sd)


def _pass_c(tables, pk, csplat, zstripe):
    P = len(tables) + 1
    NCH = EPW // CA
    mesh = plsc.VectorSubcoreMesh(core_axis_name="c", subcore_axis_name="s")

    @functools.partial(
        pl.kernel,
        out_type=tuple(jax.ShapeDtypeStruct((NC, NPAD, 32), jnp.float32)
                       for _ in range(P)),
        mesh=mesh,
        scratch_types=(
            pltpu.VMEM((3, CA), jnp.int32),
            pltpu.VMEM((3, CA), jnp.int32),
            pltpu.VMEM((3, CA), jnp.int32),
            pltpu.VMEM((3, CA), jnp.int32),
            pltpu.VMEM((CA, 32), jnp.float32),
            pltpu.VMEM((CA, 32), jnp.float32),
            pltpu.VMEM((CA,), jnp.float32),
            pltpu.VMEM((16,), jnp.float32),
            pltpu.VMEM_SHARED((NPAD, 32), jnp.float32),
            pltpu.SemaphoreType.DMA,
            pltpu.SemaphoreType.DMA,
            pltpu.SemaphoreType.DMA,
            pltpu.SemaphoreType.DMA,
            pltpu.SemaphoreType.DMA,
            pltpu.SemaphoreType.DMA,
            pltpu.SemaphoreType.DMA,
            pltpu.SemaphoreType.DMA,
        ),
        compiler_params=_SC_PARAMS,
    )
    def k(*refs):
        t_hs = refs[:P - 1]
        pk_h, c_h, z_h = refs[P - 1:P + 2]
        out_hs = refs[P + 2:2 * P + 2]
        (pb0, pb1, pb2, pb3, rows0, rows1, abuf, cv, slab,
         st0, st1, st2, st3, sg0, sg1, sc0, sc1) = refs[2 * P + 2:]
        c = lax.axis_index("c")
        s = lax.axis_index("s")
        wid = s * NC + c
        pltpu.sync_copy(c_h, cv)
        cvec = cv[...]
        iota = lax.iota(jnp.int32, 16)
        pbs = (pb0, pb1, pb2, pb3)
        sts = (st0, st1, st2, st3)
        rws = (rows0, rows1)
        sgs = (sg0, sg1)
        scs = (sc0, sc1)

        for p in range(P):
            is_den = p == P - 1
            pltpu.sync_copy(z_h, slab.at[pl.ds(s * STRIPE, STRIPE)])
            plsc.subcore_barrier()

            def stage(i, q):
                pltpu.async_copy(pk_h.at[wid * NCH + i], pbs[q], sts[q])

            def stage_wait(q):
                pltpu.make_async_copy(pk_h.at[wid * NCH], pbs[q], sts[q]).wait()

            def gissue(i, q, pa):
                pltpu.async_copy(t_hs[p].at[pbs[q].at[0]], rws[pa], sgs[pa])

            def gwait(q, pa):
                pltpu.make_async_copy(
                    t_hs[p].at[pbs[q].at[0]], rws[pa], sgs[pa]).wait()

            def scwait(q, pa):
                pltpu.make_async_copy(
                    rws[pa], slab.at[pbs[q].at[1]], scs[pa]).wait()

            stage(0, 0)
            stage(1, 1)
            stage_wait(0)
            if not is_den:
                gissue(0, 0, 0)

            def outer(cj, _):
                for u in range(4):
                    i4 = 4 * cj + u
                    q = u
                    pa = u % 2
                    qn = (u + 1) % 4
                    qnn = (u + 2) % 4
                    nxt = i4 + 1

                    @pl.when(nxt < NCH)
                    def _():
                        stage_wait(qn)

                    @pl.when(i4 >= 1)
                    def _():
                        scwait((u - 1) % 4, 1 - pa)

                    if not is_den:
                        @pl.when(nxt < NCH)
                        def _():
                            gissue(nxt, qn, 1 - pa)

                        gwait(q, pa)

                    @pl.when(i4 + 2 < NCH)
                    def _():
                        stage(i4 + 2, qnn)

                    pb = pbs[q]
                    rows = rws[pa]

                    def agrp(g, _):
                        lv = lax.bitcast_convert_type(
                            pb[2, pl.ds(g * 16, 16)], jnp.float32)
                        a = jnp.exp(lv - cvec)
                        abuf[pl.ds(g * 16, 16)] = a
                        return 0

                    lax.fori_loop(0, CA // 16, agrp, 0)

                    if is_den:
                        def dgrp(g, _):
                            av = abuf[pl.ds(g * 16, 16)]
                            for e in range(16):
                                spl = av[jnp.full((16,), e, jnp.int32)]
                                rows[g * 16 + e, pl.ds(0, 16)] = \
                                    jnp.where(iota == 0, spl, 0.0)
                                rows[g * 16 + e, pl.ds(16, 16)] = \
                                    jnp.zeros((16,), jnp.float32)
                            return 0

                        lax.fori_loop(0, CA // 16, dgrp, 0)
                    else:
                        def sgrp(g, _):
                            av = abuf[pl.ds(g * 16, 16)]
                            for e in range(16):
                                spl = av[jnp.full((16,), e, jnp.int32)]
                                r = g * 16 + e
                                rows[r, pl.ds(0, 16)] = rows[r, pl.ds(0, 16)] * spl
                                rows[r, pl.ds(16, 16)] = rows[r, pl.ds(16, 16)] * spl
                            return 0

                        lax.fori_loop(0, CA // 16, sgrp, 0)

                    pltpu.async_copy(rows, slab.at[pb.at[1]], scs[pa], add=True)
                return 0

            lax.fori_loop(0, NCH // 4, outer, 0)
            scwait((NCH - 1) % 4, (NCH - 1) % 2)
            plsc.subcore_barrier()
            pltpu.sync_copy(slab.at[pl.ds(s * STRIPE, STRIPE)],
                            out_hs[p].at[c, pl.ds(s * STRIPE, STRIPE)])

    return k(*tables, pk, csplat, zstripe)


def _selflogit_kernel(xl_ref, xr_ref, att_ref, o_ref, m_ref):
    t = xl_ref[...] + xr_ref[...]
    t = jnp.maximum(t, 0.2 * t)
    sl = t @ att_ref[...]
    o_ref[...] = sl

    @pl.when(pl.program_id(0) == 0)
    def _():
        m_ref[...] = jnp.full((1, 128), NEG, jnp.float32)

    m_ref[...] = jnp.maximum(m_ref[...], jnp.max(sl))


def _selflogit(xl, xr, att, block_m=512):
    M, D = xl.shape
    Mp = ((M + block_m - 1) // block_m) * block_m
    if Mp != M:
        xl = jnp.pad(xl, ((0, Mp - M), (0, 0)))
        xr = jnp.pad(xr, ((0, Mp - M), (0, 0)))
    sl, m = pl.pallas_call(
        _selflogit_kernel,
        grid=(Mp // block_m,),
        in_specs=[
            pl.BlockSpec((block_m, D), lambda i: (i, 0)),
            pl.BlockSpec((block_m, D), lambda i: (i, 0)),
            pl.BlockSpec((D,), lambda i: (0,)),
        ],
        out_specs=[
            pl.BlockSpec((block_m,), lambda i: (i,)),
            pl.BlockSpec((1, 128), lambda i: (0, 0)),
        ],
        out_shape=[
            jax.ShapeDtypeStruct((Mp,), jnp.float32),
            jax.ShapeDtypeStruct((1, 128), jnp.float32),
        ],
    )(xl, xr, att)
    return sl[:M], m


def _pool_kernel(h_ref, b_ref, s_ref, c_ref):
    oh = (b_ref[...][:, None] == lax.iota(jnp.int32, B)[None, :]).astype(jnp.float32)

    @pl.when(pl.program_id(0) == 0)
    def _():
        s_ref[...] = jnp.zeros_like(s_ref)
        c_ref[...] = jnp.zeros_like(c_ref)

    s_ref[...] += jnp.dot(oh.T, h_ref[...], preferred_element_type=jnp.float32)
    c_ref[...] += jnp.sum(oh, axis=0)[None, :]


def _pool(h, batch, block_m=512):
    M = h.shape[0]
    Mp = ((M + block_m - 1) // block_m) * block_m
    if Mp != M:
        h = jnp.pad(h, ((0, Mp - M), (0, 0)))
        batch = jnp.pad(batch, (0, Mp - M), constant_values=B)
    sums, counts = pl.pallas_call(
        _pool_kernel,
        grid=(Mp // block_m,),
        in_specs=[
            pl.BlockSpec((block_m, 32), lambda i: (i, 0)),
            pl.BlockSpec((block_m,), lambda i: (i,)),
        ],
        out_specs=[
            pl.BlockSpec((B, 32), lambda i: (0, 0)),
            pl.BlockSpec((1, B), lambda i: (0, 0)),
        ],
        out_shape=[
            jax.ShapeDtypeStruct((B, 32), jnp.float32),
            jax.ShapeDtypeStruct((1, B), jnp.float32),
        ],
    )(h, batch)
    return sums / jnp.maximum(counts[0], 1.0)[:, None]


def _merge_kernel(xl_ref, sl_ref, cb_ref, b_ref, *rest, nq):
    o_refs = rest[:nq + 1]
    h_ref = rest[nq + 1]
    C = cb_ref[0, 0]
    sw = jnp.exp(sl_ref[...] - C)
    num = jnp.concatenate(
        [o[0] + o[1] for o in (r[...] for r in o_refs[:nq])], axis=1)
    den = o_refs[nq][...][0, :, 0] + o_refs[nq][...][1, :, 0]
    h = (num + sw[:, None] * xl_ref[...]) / (den + sw)[:, None] + b_ref[...][None, :]
    h_ref[...] = jnp.where(h > 0, h, jnp.exp(jnp.minimum(h, 0.0)) - 1.0)


def _merge(outs, xl, sl, C, bias, D, block_m=2048):
    nq = D // 32
    M = xl.shape[0]
    Mp = ((M + block_m - 1) // block_m) * block_m
    if Mp != M:
        xl = jnp.pad(xl, ((0, Mp - M), (0, 0)))
        sl = jnp.pad(sl, (0, Mp - M))
    cb = jnp.full((1, 128), C, jnp.float32)
    h = pl.pallas_call(
        functools.partial(_merge_kernel, nq=nq),
        grid=(Mp // block_m,),
        in_specs=[
            pl.BlockSpec((block_m, D), lambda i: (i, 0)),
            pl.BlockSpec((block_m,), lambda i: (i,)),
            pl.BlockSpec((1, 128), lambda i: (0, 0)),
            pl.BlockSpec((D,), lambda i: (0,)),
        ] + [pl.BlockSpec((2, block_m, 32), lambda i: (0, i, 0))
             for _ in range(nq + 1)],
        out_specs=pl.BlockSpec((block_m, D), lambda i: (i, 0)),
        out_shape=jax.ShapeDtypeStruct((Mp, D), jnp.float32),
    )(xl, sl, cb, bias, *outs)
    return h[:M]


def _gatv2_sc(xl, xr, att, bias, sd, srcp, dstp, zstripe, D):
    lo, wmax = _pass_a(xl, xr, att, sd, D)
    sl, smax = _selflogit(xl, xr, att)
    C = jnp.maximum(jnp.max(wmax), jnp.max(smax))
    csplat = jnp.full((16,), C, jnp.float32)
    pk = jnp.concatenate([
        srcp.reshape(-1, 1, CA),
        dstp.reshape(-1, 1, CA),
        lax.bitcast_convert_type(lo, jnp.int32).reshape(-1, 1, CA),
    ], axis=1)
    nq = D // 32
    tables = [xl[:, q * 32:(q + 1) * 32] for q in range(nq)]
    outs = _pass_c(tables, pk, csplat, zstripe)
    return _merge(outs, xl, sl, C, bias, D)


def kernel(x, edge_index, batch, observable_features, noise_factor, noisy_exp, Wl1, bl1, Wr1, br1, att1, bias1, Wl2, bl2, Wr2, br2, att2, bias2, Wo1, bo1, Wo2, bo2, Wn, bn, Wf1, bf1, Wf2, bf2, Wf3, bf3):
    srcp = jnp.pad(edge_index[0], (0, EPAD - E))
    dstp = jnp.pad(edge_index[1], (0, EPAD - E))
    sd = jnp.concatenate(
        [srcp.reshape(-1, 1, CA), dstp.reshape(-1, 1, CA)], axis=1)
    zstripe = jnp.zeros((STRIPE, 32), jnp.float32)

    xl1 = _mm(x, Wl1, bl1)
    xr1 = _mm(x, Wr1, br1)
    h = _gatv2_sc(xl1, xr1, att1, bias1, sd, srcp, dstp, zstripe, 128)
    xl2 = _mm(h, Wl2, bl2)
    xr2 = _mm(h, Wr2, br2)
    h = _gatv2_sc(xl2, xr2, att2, bias2, sd, srcp, dstp, zstripe, 32)

    circuit_embedding = _pool(h, batch)
    obs_embedding = _mm(_mm(observable_features, Wo1, bo1, act="relu"), Wo2, bo2, block_m=64)
    noise_embedding = _mm(noise_factor, Wn, bn, block_m=64)
    ne = noisy_exp.reshape(-1, 1)
    combined = jnp.concatenate([circuit_embedding, obs_embedding, noise_embedding, ne], axis=1)
    c = _mm(combined, Wf1, bf1, act="relu", block_m=64)
    c = _mm(c, Wf2, bf2, act="relu", block_m=64)
    correction = _mm(c, Wf3, bf3, block_m=64)
    return ne + correction
